# baseline (speedup 1.0000x reference)
import functools

import jax
import jax.numpy as jnp
from jax.experimental import pallas as pl
from jax.experimental.pallas import tpu as pltpu

N = 10000
E = 160000
D = 128
DE = 16
H = 256
HE = 128
L = 4
V = 1000
B = 256


def _mm_body(a_ref, b_ref, o_ref):
    o_ref[...] = jnp.dot(a_ref[...], b_ref[...],
                         preferred_element_type=jnp.float32)


def _matmul(a, b, bm=512):
    M, K = a.shape
    K2, Nc = b.shape
    assert K == K2
    Mp = ((M + bm - 1) // bm) * bm
    if Mp != M:
        a = jnp.pad(a, ((0, Mp - M), (0, 0)))
    out = pl.pallas_call(
        _mm_body,
        grid=(Mp // bm,),
        in_specs=[pl.BlockSpec((bm, K), lambda i: (i, 0)),
                  pl.BlockSpec((K, Nc), lambda i: (0, 0))],
        out_specs=pl.BlockSpec((bm, Nc), lambda i: (i, 0)),
        out_shape=jax.ShapeDtypeStruct((Mp, Nc), jnp.float32),
    )(a, b)
    return out[:M] if Mp != M else out


def _mlp(p, x):
    return jnp.maximum(_matmul(x, p['W1']) + p['b1'], 0.0) @ p['W2'] + p['b2']


def _encoder(p, x_node, x_edge, src, dst):
    h = jnp.maximum(_matmul(x_node, p['W_in']) + p['b_in'], 0.0)
    he = jnp.maximum(_matmul(x_edge, p['W_e_in']) + p['b_e_in'], 0.0)
    agg_eh = _matmul(jax.ops.segment_sum(he, dst, num_segments=N), p['W_eh'])
    for l in range(L):
        hm = _matmul(h, p['W_msg'][l])
        agg = jax.ops.segment_sum(hm[src], dst, num_segments=N) + agg_eh
        h = jnp.maximum(_matmul(h, p['W_self'][l]) + agg + p['b'][l], 0.0)
    return h


def _set2set(p, h_node, batch):
    q_star = jnp.zeros((B, 2 * H), dtype=h_node.dtype)
    h0 = jnp.zeros((B, H), dtype=h_node.dtype)
    c0 = h0
    h1 = h0
    c1 = h0
    for _ in range(6):
        g = q_star @ p['W_ih0'].T + h0 @ p['W_hh0'].T + p['b0']
        i, f, gg, o = jnp.split(g, 4, axis=-1)
        c0 = jax.nn.sigmoid(f) * c0 + jax.nn.sigmoid(i) * jnp.tanh(gg)
        h0 = jax.nn.sigmoid(o) * jnp.tanh(c0)
        g = h0 @ p['W_ih1'].T + h1 @ p['W_hh1'].T + p['b1']
        i, f, gg, o = jnp.split(g, 4, axis=-1)
        c1 = jax.nn.sigmoid(f) * c1 + jax.nn.sigmoid(i) * jnp.tanh(gg)
        h1 = jax.nn.sigmoid(o) * jnp.tanh(c1)
        q = h1
        e = jnp.sum(h_node * q[batch], axis=-1)
        emax = jax.ops.segment_max(e, batch, num_segments=B)
        emax = jnp.where(jnp.isfinite(emax), emax, 0.0)
        ex = jnp.exp(e - emax[batch])
        denom = jax.ops.segment_sum(ex, batch, num_segments=B)
        a = ex / (denom[batch] + 1e-12)
        r = jax.ops.segment_sum(a[:, None] * h_node, batch, num_segments=B)
        q_star = jnp.concatenate([q, r], axis=-1)
    return q_star


def kernel(x_node, x_edge, edge_index, node2graph, params):
    src = edge_index[0]
    dst = edge_index[1]
    h_act = _encoder(params['enc_act'], x_node, x_edge, src, dst)
    h_del = _encoder(params['enc_del'], x_node, x_edge, src, dst)
    h_add = _encoder(params['enc_add'], x_node, x_edge, src, dst)
    h_arm = _encoder(params['enc_arm'], x_node, x_edge, src, dst)
    q = _set2set(params['s2s'], h_act, node2graph)
    pred_act = _mlp(params['cls_act'], q)
    h_edge = _mlp(params['edge_mlp'], x_edge)
    pred_del = _mlp(params['cls_del'],
                    jnp.concatenate([h_del[src], h_edge, h_del[dst]], axis=1))
    pred_add = _mlp(params['cls_add'], h_add)
    pred_arm = _mlp(params['cls_arm'], h_arm)
    return (pred_act, pred_del, pred_add, pred_arm)
